# baseline (speedup 1.0000x reference)
import functools

import jax
import jax.numpy as jnp
from jax.experimental import pallas as pl
from jax.experimental.pallas import tpu as pltpu

_NB_KNN = (10, 20, 100, 200)
_MAX_K = 200
_TEMP = 0.07
_NUM_CLASSES = 1000


def _matmul_body(f_ref, t_ref, o_ref):
    o_ref[...] = jax.lax.dot_general(
        f_ref[...], t_ref[...],
        dimension_numbers=(((1,), (1,)), ((), ())),
        preferred_element_type=jnp.float32,
    )


def _similarity(features_rank, train_features):
    q, d = features_rank.shape
    n = train_features.shape[0]
    nb = 2048
    num_blocks = pl.cdiv(n, nb)
    out = pl.pallas_call(
        _matmul_body,
        grid=(num_blocks,),
        in_specs=[
            pl.BlockSpec((q, d), lambda i: (0, 0)),
            pl.BlockSpec((nb, d), lambda i: (i, 0)),
        ],
        out_specs=pl.BlockSpec((q, nb), lambda i: (0, i)),
        out_shape=jax.ShapeDtypeStruct((q, n), jnp.float32),
    )(features_rank, train_features)
    return out


def kernel(features_rank, train_features, train_labels):
    similarity = _similarity(features_rank, train_features)
    topk_sims, indices = jax.lax.top_k(similarity, _MAX_K)
    neighbors_labels = jnp.take(train_labels, indices)
    batch_size = neighbors_labels.shape[0]
    topk_sims_transform = jax.nn.softmax(topk_sims / _TEMP, axis=1)
    one_hot = jax.nn.one_hot(neighbors_labels, _NUM_CLASSES, dtype=topk_sims.dtype)
    matmul = one_hot * topk_sims_transform.reshape(batch_size, -1, 1)
    return tuple(jnp.sum(matmul[:, :k, :], axis=1) for k in _NB_KNN)


# speedup vs baseline: 2.0507x; 2.0507x over previous
import jax
import jax.numpy as jnp
from jax.experimental import pallas as pl
from jax.experimental.pallas import tpu as pltpu

_NB_KNN = (10, 20, 100, 200)
_MAX_K = 200
_TEMP = 0.07
_NUM_CLASSES = 1000
_CCHUNK = 128
_QB = 256


def _matmul_body(f_ref, t_ref, o_ref):
    o_ref[...] = jax.lax.dot_general(
        f_ref[...], t_ref[...],
        dimension_numbers=(((1,), (1,)), ((), ())),
        preferred_element_type=jnp.float32,
    )


def _similarity(features_rank, train_features):
    q, d = features_rank.shape
    n = train_features.shape[0]
    nb = 2048
    num_blocks = pl.cdiv(n, nb)
    return pl.pallas_call(
        _matmul_body,
        grid=(num_blocks,),
        in_specs=[
            pl.BlockSpec((q, d), lambda i: (0, 0)),
            pl.BlockSpec((nb, d), lambda i: (i, 0)),
        ],
        out_specs=pl.BlockSpec((q, nb), lambda i: (0, i)),
        out_shape=jax.ShapeDtypeStruct((q, n), jnp.float32),
    )(features_rank, train_features)


def _vote_body(s_ref, l_ref, o10, o20, o100, o200, w_ref):
    s = s_ref[...] * (1.0 / _TEMP)
    m = jnp.max(s, axis=0, keepdims=True)
    e = jnp.exp(s - m)
    w_ref[...] = e / jnp.sum(e, axis=0, keepdims=True)
    qb = s.shape[1]
    outs = {10: o10, 20: o20, 100: o100, 200: o200}
    for cls0 in range(0, _NUM_CLASSES, _CCHUNK):
        csz = min(_CCHUNK, _NUM_CLASSES - cls0)
        cid = jax.lax.broadcasted_iota(jnp.int32, (csz, 1), 0) + cls0

        def step(j, acc):
            lab_j = l_ref[pl.ds(j, 1), :]
            w_j = w_ref[pl.ds(j, 1), :]
            return acc + jnp.where(cid == lab_j, w_j, 0.0)

        acc = jnp.zeros((csz, qb), jnp.float32)
        prev = 0
        for k in _NB_KNN:
            acc = jax.lax.fori_loop(prev, k, step, acc)
            outs[k][pl.ds(cls0, csz), :] = acc
            prev = k


def _vote(topk_sims, labels):
    q, k = topk_sims.shape
    grid = (q // _QB,)
    outs = pl.pallas_call(
        _vote_body,
        grid=grid,
        in_specs=[
            pl.BlockSpec((k, _QB), lambda i: (0, i)),
            pl.BlockSpec((k, _QB), lambda i: (0, i)),
        ],
        out_specs=[pl.BlockSpec((_NUM_CLASSES, _QB), lambda i: (0, i))] * 4,
        out_shape=[jax.ShapeDtypeStruct((_NUM_CLASSES, q), jnp.float32)] * 4,
        scratch_shapes=[pltpu.VMEM((k, _QB), jnp.float32)],
    )(topk_sims.T, labels.T)
    return tuple(o.T for o in outs)


def kernel(features_rank, train_features, train_labels):
    similarity = _similarity(features_rank, train_features)
    q, n = similarity.shape
    chunk = 2000
    if n % chunk == 0 and n > 2 * chunk:
        nchunks = n // chunk
        sim3 = similarity.reshape(q, nchunks, chunk)
        v1, i1 = jax.lax.top_k(sim3, _MAX_K)
        base = (jnp.arange(nchunks, dtype=i1.dtype) * chunk)[None, :, None]
        g1 = (i1 + base).reshape(q, nchunks * _MAX_K)
        v1f = v1.reshape(q, nchunks * _MAX_K)
        topk_sims, i2 = jax.lax.top_k(v1f, _MAX_K)
        indices = jnp.take_along_axis(g1, i2, axis=1)
    else:
        topk_sims, indices = jax.lax.top_k(similarity, _MAX_K)
    neighbors_labels = jnp.take(train_labels, indices)
    return _vote(topk_sims, neighbors_labels)
